# Initial kernel scaffold; baseline (speedup 1.0000x reference)
import jax
import jax.numpy as jnp
from jax.experimental import pallas as pl


def kernel(x, w_lin, b_lin, w1r, w1i, w2r, w2i):
    raise NotImplementedError("write your pallas kernel here")


# speedup vs baseline: 2.8853x; 2.8853x over previous
import math

import numpy as np
import jax
import jax.numpy as jnp
from jax.experimental import pallas as pl
from jax.experimental.pallas import tpu as pltpu

_INV_SQRT2 = 1.0 / math.sqrt(2.0)


def _dft_consts(H, W, m1, m2):
    k1 = np.concatenate([np.arange(m1), np.arange(H - m1, H)]).astype(np.float64)
    k2 = np.arange(m2, dtype=np.float64)
    h = np.arange(H, dtype=np.float64)
    w = np.arange(W, dtype=np.float64)

    fscale = 1.0 / math.sqrt(H * W)

    ang_fh = 2.0 * np.pi * np.outer(k1, h) / H
    fhr = np.cos(ang_fh) * fscale
    fhi = -np.sin(ang_fh) * fscale
    ang_fw = 2.0 * np.pi * np.outer(w, k2) / W
    gfr = np.cos(ang_fw)
    gfi = -np.sin(ang_fw)

    ang_eh = 2.0 * np.pi * np.outer(h, k1) / H
    ehr = np.cos(ang_eh)
    ehi = np.sin(ang_eh)
    c = np.where(np.logical_or(k2 == 0,
                               np.logical_and(W % 2 == 0, k2 == (W // 2))),
                 1.0, 2.0)
    ang_gw = 2.0 * np.pi * np.outer(k2, w) / W
    iscale = 1.0 / (H * W)
    gwr = np.cos(ang_gw) * c[:, None] * iscale
    gwi = np.sin(ang_gw) * c[:, None] * iscale

    G = np.concatenate([gfr, gfi], axis=1)
    FHT = np.concatenate([fhr, fhi], axis=0).T
    EH_r = np.concatenate([ehr.T, -ehi.T], axis=0)
    EH_i = np.concatenate([ehi.T, ehr.T], axis=0)
    Gw_cat = np.concatenate([gwr, -gwi], axis=0)

    f32 = lambda a: jnp.asarray(a, dtype=jnp.float32)
    return f32(G), f32(FHT), f32(EH_r), f32(EH_i), f32(Gw_cat)


def _fused_kernel(x_ref, g_ref, fht_ref, ehr_ref, ehi_ref, gw_ref,
                  wr_ref, wsum_ref, wdif_ref, wlin_ref, b_ref, o_ref,
                  *, m1, m2):
    ci, H, W = x_ref.shape[1], x_ref.shape[2], x_ref.shape[3]
    co = o_ref.shape[1]
    two_m1 = 2 * m1
    K = m2 * two_m1

    xb = x_ref[0]

    A = jnp.dot(xb.reshape(ci * H, W), g_ref[...],
                preferred_element_type=jnp.float32)
    A = A.reshape(ci, H, 2 * m2).transpose(0, 2, 1)
    P = jnp.dot(A.reshape(ci * 2 * m2, H), fht_ref[...],
                preferred_element_type=jnp.float32)
    P = P.reshape(ci, 2 * m2, 2 * two_m1)

    xr = P[:, :m2, :two_m1] - P[:, m2:, two_m1:]
    xi = P[:, m2:, :two_m1] + P[:, :m2, two_m1:]

    xr = xr.reshape(ci, K)
    xi = xi.reshape(ci, K)

    kt = wr_ref[...] * (xr + xi)[:, None, :]
    yr = jnp.sum(kt - xi[:, None, :] * wsum_ref[...], axis=0)
    yi = jnp.sum(kt + xr[:, None, :] * wdif_ref[...], axis=0)

    ycat = jnp.concatenate([yr.reshape(co, m2, two_m1),
                            yi.reshape(co, m2, two_m1)], axis=2)
    ycat2 = ycat.reshape(co * m2, 2 * two_m1)
    pr = jnp.dot(ycat2, ehr_ref[...],
                 preferred_element_type=jnp.float32)
    pi = jnp.dot(ycat2, ehi_ref[...],
                 preferred_element_type=jnp.float32)
    pboth = jnp.concatenate([pr.reshape(co, m2, H),
                             pi.reshape(co, m2, H)], axis=1)
    pboth = pboth.transpose(0, 2, 1)
    y_spec = jnp.dot(pboth.reshape(co * H, 2 * m2), gw_ref[...],
                     preferred_element_type=jnp.float32)
    y_spec = y_spec.reshape(co, H, W)

    res = jnp.dot(wlin_ref[...], xb.reshape(ci, H * W),
                  preferred_element_type=jnp.float32)
    z = y_spec + res.reshape(co, H, W) + b_ref[...][:, :, None]
    o_ref[0] = 0.5 * z * (1.0 + jax.lax.erf(z * _INV_SQRT2))


def kernel(x, w_lin, b_lin, w1r, w1i, w2r, w2i):
    B, Ci, H, W = x.shape
    Co = w_lin.shape[0]
    m1, m2 = w1r.shape[2], w1r.shape[3]
    two_m1 = 2 * m1
    K = m2 * two_m1

    G, FHT, EH_r, EH_i, Gw_cat = _dft_consts(H, W, m1, m2)

    wr = jnp.concatenate([w1r, w2r], axis=2).astype(jnp.float32)
    wi = jnp.concatenate([w1i, w2i], axis=2).astype(jnp.float32)
    wr_m = wr.transpose(0, 1, 3, 2).reshape(Ci, Co, K)
    wi_m = wi.transpose(0, 1, 3, 2).reshape(Ci, Co, K)
    wsum = wr_m + wi_m
    wdif = wi_m - wr_m

    x32 = x.astype(jnp.float32)
    wlin32 = w_lin.astype(jnp.float32)
    blin32 = b_lin.astype(jnp.float32)

    const = lambda b: (0, 0)
    const3 = lambda b: (0, 0, 0)

    from functools import partial
    out = pl.pallas_call(
        partial(_fused_kernel, m1=m1, m2=m2),
        out_shape=jax.ShapeDtypeStruct((B, Co, H, W), jnp.float32),
        grid=(B,),
        in_specs=[
            pl.BlockSpec((1, Ci, H, W), lambda b: (b, 0, 0, 0)),
            pl.BlockSpec((W, 2 * m2), const),
            pl.BlockSpec((H, 2 * two_m1), const),
            pl.BlockSpec((2 * two_m1, H), const),
            pl.BlockSpec((2 * two_m1, H), const),
            pl.BlockSpec((2 * m2, W), const),
            pl.BlockSpec((Ci, Co, K), const3),
            pl.BlockSpec((Ci, Co, K), const3),
            pl.BlockSpec((Ci, Co, K), const3),
            pl.BlockSpec((Co, Ci), const),
            pl.BlockSpec((Co, 1), const),
        ],
        out_specs=pl.BlockSpec((1, Co, H, W), lambda b: (b, 0, 0, 0)),
        compiler_params=pltpu.CompilerParams(
            dimension_semantics=("parallel",)),
    )(x32, G, FHT, EH_r, EH_i, Gw_cat, wr_m, wsum, wdif, wlin32, blin32)
    return out
